# Initial kernel scaffold; baseline (speedup 1.0000x reference)
import jax
import jax.numpy as jnp
from jax.experimental import pallas as pl


def kernel(event_ids, query_embeddings, template_table):
    raise NotImplementedError("write your pallas kernel here")


# speedup vs baseline: 1.0952x; 1.0952x over previous
import functools

import jax
import jax.numpy as jnp
from jax import lax
from jax.experimental import pallas as pl
from jax.experimental.pallas import tpu as pltpu
from jax.experimental.pallas import tpu_sc as plsc

_NUM_CLASSES = 100000
_D = 128
_B = 1024
_EPS = 1e-6

_BK = 2000
_NKB = _NUM_CLASSES // _BK
_BIG = 2**30


def _nearest_body(q_ref, keys_ref, out_ref, qn_ref, bval_ref, bidx_ref):
    kb = pl.program_id(0)

    @pl.when(kb == 0)
    def _init():
        q = q_ref[...]
        qsq = lax.dot_general(
            jnp.ones((1, _D), jnp.float32), q * q,
            (((1,), (1,)), ((), ())), preferred_element_type=jnp.float32,
            precision=lax.Precision.HIGHEST)
        qn_ref[...] = jnp.sqrt(qsq)
        bval_ref[...] = jnp.full((1, _B), -jnp.inf, jnp.float32)
        bidx_ref[...] = jnp.zeros((1, _B), jnp.int32)

    keys = keys_ref[...]
    dots = lax.dot_general(
        keys.astype(jnp.bfloat16), q_ref[...].astype(jnp.bfloat16),
        (((1,), (1,)), ((), ())),
        preferred_element_type=jnp.float32)
    kn = jnp.sqrt(jnp.sum(keys * keys, axis=1, keepdims=True))
    sims = dots / jnp.maximum(kn * qn_ref[...], _EPS)

    lmax = jnp.max(sims, axis=0, keepdims=True)
    rows = kb * _BK + lax.broadcasted_iota(jnp.int32, (_BK, _B), 0)
    lidx = jnp.min(jnp.where(sims == lmax, rows, _BIG),
                   axis=0, keepdims=True)
    better = lmax > bval_ref[...]
    bval_ref[...] = jnp.where(better, lmax, bval_ref[...])
    bidx_ref[...] = jnp.where(better, lidx, bidx_ref[...])

    @pl.when(kb == _NKB - 1)
    def _done():
        out_ref[...] = bidx_ref[...]


def _nearest_tc(query_embeddings, template_table, interpret=False):
    return pl.pallas_call(
        _nearest_body,
        grid=(_NKB,),
        in_specs=[
            pl.BlockSpec((_B, _D), lambda kb: (0, 0)),
            pl.BlockSpec((_BK, _D), lambda kb: (kb, 0)),
        ],
        out_specs=pl.BlockSpec((1, _B), lambda kb: (0, 0)),
        out_shape=jax.ShapeDtypeStruct((1, _B), jnp.int32),
        scratch_shapes=[
            pltpu.VMEM((1, _B), jnp.float32),
            pltpu.VMEM((1, _B), jnp.float32),
            pltpu.VMEM((1, _B), jnp.int32),
        ],
        compiler_params=pltpu.CompilerParams(
            dimension_semantics=("arbitrary",)),
        interpret=interpret,
    )(query_embeddings, template_table)


_NC = 2
_NS = 16
_L = 16
_NW = _NC * _NS
_BPW = _B // _NW


@functools.cache
def _make_gather():
    @functools.partial(
        pl.kernel,
        out_type=jax.ShapeDtypeStruct((_B, _D), jnp.float32),
        mesh=plsc.VectorSubcoreMesh(core_axis_name="c", subcore_axis_name="s"),
        scratch_types=[
            pltpu.VMEM((_BPW,), jnp.int32),
            pltpu.VMEM((_BPW,), jnp.int32),
            pltpu.VMEM((_BPW,), jnp.int32),
            pltpu.VMEM((_BPW, _D), jnp.float32),
            pltpu.SemaphoreType.DMA,
        ],
    )
    def _gather_kernel(table_hbm, ev_hbm, near_hbm, out_hbm,
                       ev_v, near_v, idx_v, rows_v, sem):
        wid = lax.axis_index("s") * _NC + lax.axis_index("c")
        base = wid * _BPW
        pltpu.sync_copy(ev_hbm.at[pl.ds(base, _BPW)], ev_v)
        pltpu.sync_copy(near_hbm.at[pl.ds(base, _BPW)], near_v)
        for i in range(_BPW // _L):
            sl = pl.ds(i * _L, _L)
            ev = ev_v[sl]
            idx_v[sl] = jnp.where(ev > _NUM_CLASSES, near_v[sl], ev)
        pltpu.async_copy(table_hbm.at[idx_v], rows_v, sem).wait()
        pltpu.sync_copy(rows_v, out_hbm.at[pl.ds(base, _BPW)])

    return _gather_kernel


def kernel(event_ids, query_embeddings, template_table):
    nearest = _nearest_tc(query_embeddings, template_table)
    return _make_gather()(template_table, event_ids, nearest.reshape(_B))
